# baseline (speedup 1.0000x reference)
import functools

import jax
import jax.numpy as jnp
from jax import lax
from jax.experimental import pallas as pl
from jax.experimental.pallas import tpu as pltpu
from jax.experimental.pallas import tpu_sc as plsc

N = 8192
D = 2048
HALF = N // 2
NUM_SUBCORES = 16
ROWS = HALF // NUM_SUBCORES
R = 16
C = ROWS // R
S = 3


def _body(x, out, buf, in_sems, out_sems):
    h = lax.axis_index("c")
    t = lax.axis_index("s")
    o0 = h * HALF + t * ROWS
    lane = lax.iota(jnp.int32, 16)

    def start_in(k, slot):
        src_rows = (t * ROWS + k * R + lane) * 2 + h
        return pltpu.async_copy(x.at[src_rows], buf.at[slot], in_sems.at[slot])

    def start_out(k, slot):
        return pltpu.async_copy(
            buf.at[slot], out.at[pl.ds(o0 + k * R, R)], out_sems.at[slot]
        )

    ins = [None] * C
    outs = [None] * C
    for k in range(C):
        slot = k % S
        if k >= S:
            outs[k - S].wait()
        ins[k] = start_in(k, slot)
        if k >= 1:
            ins[k - 1].wait()
            outs[k - 1] = start_out(k - 1, (k - 1) % S)
    ins[C - 1].wait()
    outs[C - 1] = start_out(C - 1, (C - 1) % S)
    for k in range(max(C - S + 1, 0), C):
        outs[k].wait()


@jax.jit
def _shuffle(x):
    mesh = plsc.VectorSubcoreMesh(core_axis_name="c", subcore_axis_name="s")
    return pl.kernel(
        _body,
        out_type=jax.ShapeDtypeStruct((N, D), jnp.float32),
        mesh=mesh,
        scratch_types=[
            pltpu.VMEM((S, R, D), jnp.float32),
            pltpu.SemaphoreType.DMA((S,)),
            pltpu.SemaphoreType.DMA((S,)),
        ],
    )(x)


def kernel(inputs):
    return _shuffle(inputs)


# speedup vs baseline: 1.0585x; 1.0585x over previous
import functools

import jax
import jax.numpy as jnp
from jax import lax
from jax.experimental import pallas as pl
from jax.experimental.pallas import tpu as pltpu
from jax.experimental.pallas import tpu_sc as plsc

N = 8192
D = 2048
HALF = N // 2
NUM_SUBCORES = 16
ROWS = HALF // NUM_SUBCORES
R = 16
C = ROWS // R


def _body(x, out, buf, in_sems, out_sems):
    h = lax.axis_index("c")
    t = lax.axis_index("s")
    o0 = h * HALF + t * ROWS
    lane = lax.iota(jnp.int32, 16)

    def in_desc(k, slot):
        src_rows = (t * ROWS + k * R + lane) * 2 + h
        return pltpu.make_async_copy(x.at[src_rows], buf.at[slot], in_sems.at[slot])

    def out_desc(k, slot):
        return pltpu.make_async_copy(
            buf.at[slot], out.at[pl.ds(o0 + k * R, R)], out_sems.at[slot]
        )

    @pl.loop(0, C)
    def _chunk(g):
        slot = lax.rem(g, 2)

        @pl.when(g >= 2)
        def _():
            out_desc(g - 2, slot).wait()

        in_desc(g, slot).start()

        @pl.when(g >= 1)
        def _():
            in_desc(g - 1, 1 - slot).wait()
            out_desc(g - 1, 1 - slot).start()

    in_desc(C - 1, (C - 1) % 2).wait()
    out_desc(C - 1, (C - 1) % 2).start()
    out_desc(C - 2, (C - 2) % 2).wait()
    out_desc(C - 1, (C - 1) % 2).wait()


@jax.jit
def _shuffle(x):
    mesh = plsc.VectorSubcoreMesh(core_axis_name="c", subcore_axis_name="s")
    return pl.kernel(
        _body,
        out_type=jax.ShapeDtypeStruct((N, D), jnp.float32),
        mesh=mesh,
        scratch_types=[
            pltpu.VMEM((2, R, D), jnp.float32),
            pltpu.SemaphoreType.DMA((2,)),
            pltpu.SemaphoreType.DMA((2,)),
        ],
    )(x)


def kernel(inputs):
    return _shuffle(inputs)
